# Initial kernel scaffold; baseline (speedup 1.0000x reference)
import functools

import jax
import jax.numpy as jnp
from jax import lax
from jax.experimental import pallas as pl
from jax.experimental.pallas import tpu as pltpu
from jax.experimental.pallas import tpu_sc as plsc

N_CORES = 2
N_SUB = 16
N_TILES = N_CORES * N_SUB
BLK = 128
EPS = 1e-5


def _ceil_to(v, m):
    return -(-v // m) * m


@functools.lru_cache(maxsize=None)
def _make_seg_sum(n_pad, n_chunks, w):
    rpt = n_pad // N_SUB
    mesh = plsc.VectorSubcoreMesh(core_axis_name="c", subcore_axis_name="s")

    def body(table, src, dst, zeros, out, src_v, dst_v, rows_v, acc, sem):
        c = lax.axis_index("c")
        s = lax.axis_index("s")
        wid = c * N_SUB + s
        pltpu.sync_copy(src.at[wid], src_v)
        pltpu.sync_copy(dst.at[wid], dst_v)
        pltpu.sync_copy(zeros.at[pl.ds(s * rpt, rpt)], acc.at[pl.ds(s * rpt, rpt)])
        plsc.subcore_barrier()

        def step(j, carry):
            pltpu.async_copy(table.at[src_v.at[j]], rows_v, sem).wait()
            pltpu.sync_copy(rows_v, acc.at[dst_v.at[j]], add=True)
            return carry

        lax.fori_loop(0, n_chunks, step, 0)
        plsc.subcore_barrier()
        pltpu.sync_copy(acc.at[pl.ds(s * rpt, rpt)], out.at[c, pl.ds(s * rpt, rpt)])

    return pl.kernel(
        body,
        out_type=jax.ShapeDtypeStruct((N_CORES, n_pad, w), jnp.float32),
        mesh=mesh,
        scratch_types=[
            pltpu.VMEM((n_chunks, BLK), jnp.int32),
            pltpu.VMEM((n_chunks, BLK), jnp.int32),
            pltpu.VMEM((BLK, w), jnp.float32),
            pltpu.VMEM_SHARED((n_pad, w), jnp.float32),
            pltpu.SemaphoreType.DMA,
        ],
    )


def _tc_first(n, n_pad):
    def body(degp_ref, x_ref, w_ref, dinv_ref, h_ref):
        dp = degp_ref[...]
        deg = dp[0, :, 0] + dp[1, :, 0] + 1.0
        dinv = lax.rsqrt(deg)
        dinv_ref[...] = dinv[:, None]
        h = jnp.dot(x_ref[...], w_ref[...], preferred_element_type=jnp.float32)
        h_ref[...] = jnp.concatenate(
            [h * dinv[:n, None], jnp.zeros((n_pad - n, h.shape[1]), jnp.float32)], 0)
    return body


def _tc_mid(n):
    def body(p_ref, hp_ref, dinv_ref, b_ref, g_ref, be_ref, wn_ref, out_ref):
        p = p_ref[...]
        dinv = dinv_ref[...]
        u = (p[0] + p[1] + hp_ref[...]) * dinv + b_ref[...][None, :]
        n_pad = u.shape[0]
        mask = lax.broadcasted_iota(jnp.int32, (n_pad, 1), 0) < n
        u = jnp.where(mask, u, 0.0)
        mean = jnp.sum(u, axis=0, keepdims=True) / n
        var = jnp.sum(u * u, axis=0, keepdims=True) / n - mean * mean
        z = (u - mean) * lax.rsqrt(var + EPS) * g_ref[...][None, :] + be_ref[...][None, :]
        z = jnp.where(mask, jnp.maximum(z, 0.0), 0.0)
        out_ref[...] = jnp.dot(z, wn_ref[...], preferred_element_type=jnp.float32) * dinv
    return body


def _tc_last():
    def body(p_ref, hp_ref, dinv_ref, b_ref, out_ref):
        p = p_ref[...]
        out_ref[...] = (p[0] + p[1] + hp_ref[...]) * dinv_ref[...] + b_ref[...][None, :]
    return body


def _call_tc(body, out_shapes, *args):
    return pl.pallas_call(body, out_shape=out_shapes)(*args)


def kernel(x, edge_index, W1, b1, g1, beta1, W2, b2, g2, beta2,
           W3, b3, g3, beta3, W4, b4):
    n, d = x.shape
    h_dim = W1.shape[1]
    c_dim = W4.shape[1]
    e = edge_index.shape[1]

    n_chunks = -(-e // (N_TILES * BLK))
    e_pad = N_TILES * n_chunks * BLK
    n_pad = _ceil_to(n + 1, N_SUB)
    wc = _ceil_to(c_dim, 64)

    src = edge_index[0].astype(jnp.int32)
    dst = edge_index[1].astype(jnp.int32)
    fill = jnp.full((e_pad - e,), n, jnp.int32)
    src3 = jnp.concatenate([src, fill]).reshape(N_TILES, n_chunks, BLK)
    dst3 = jnp.concatenate([dst, fill]).reshape(N_TILES, n_chunks, BLK)

    zeros_h = jnp.zeros((n_pad, h_dim), jnp.float32)
    zeros_c = jnp.zeros((n_pad, wc), jnp.float32)
    zeros_16 = jnp.zeros((n_pad, 16), jnp.float32)
    ones_16 = jnp.where(jnp.arange(n_pad)[:, None] < n,
                        jnp.float32(1), jnp.float32(0)) * jnp.ones((1, 16), jnp.float32)

    seg16 = _make_seg_sum(n_pad, n_chunks, 16)
    seg_h = _make_seg_sum(n_pad, n_chunks, h_dim)
    seg_c = _make_seg_sum(n_pad, n_chunks, wc)

    degp = seg16(ones_16, src3, dst3, zeros_16)

    f32 = jnp.float32
    dinv, h1 = _call_tc(
        _tc_first(n, n_pad),
        (jax.ShapeDtypeStruct((n_pad, 1), f32), jax.ShapeDtypeStruct((n_pad, h_dim), f32)),
        degp, x, W1)

    mid = _tc_mid(n)
    p1 = seg_h(h1, src3, dst3, zeros_h)
    h2 = _call_tc(mid, jax.ShapeDtypeStruct((n_pad, h_dim), f32),
                  p1, h1, dinv, b1, g1, beta1, W2)
    p2 = seg_h(h2, src3, dst3, zeros_h)
    h3 = _call_tc(mid, jax.ShapeDtypeStruct((n_pad, h_dim), f32),
                  p2, h2, dinv, b2, g2, beta2, W3)
    p3 = seg_h(h3, src3, dst3, zeros_h)
    W4p = jnp.pad(W4, ((0, 0), (0, wc - c_dim)))
    b4p = jnp.pad(b4, (0, wc - c_dim))
    h4 = _call_tc(mid, jax.ShapeDtypeStruct((n_pad, wc), f32),
                  p3, h3, dinv, b3, g3, beta3, W4p)
    p4 = seg_c(h4, src3, dst3, zeros_c)
    outp = _call_tc(_tc_last(), jax.ShapeDtypeStruct((n_pad, wc), f32),
                    p4, h4, dinv, b4p)
    return outp[:n, :c_dim]


# speedup vs baseline: 10.0721x; 10.0721x over previous
import functools

import jax
import jax.numpy as jnp
from jax import lax
from jax.experimental import pallas as pl
from jax.experimental.pallas import tpu as pltpu
from jax.experimental.pallas import tpu_sc as plsc

N_CORES = 2
N_SUB = 16
N_TILES = N_CORES * N_SUB
BLK = 128
EPS = 1e-5


def _ceil_to(v, m):
    return -(-v // m) * m


@functools.lru_cache(maxsize=None)
def _make_seg_sum(n_pad, n_chunks, w):
    rpt = n_pad // N_SUB
    mesh = plsc.VectorSubcoreMesh(core_axis_name="c", subcore_axis_name="s")

    def body(table, src, dst, zeros, out, src_v, dst_v, rows_v, acc, sem):
        c = lax.axis_index("c")
        s = lax.axis_index("s")
        wid = c * N_SUB + s
        pltpu.sync_copy(src.at[wid], src_v)
        pltpu.sync_copy(dst.at[wid], dst_v)
        pltpu.sync_copy(zeros.at[pl.ds(s * rpt, rpt)], acc.at[pl.ds(s * rpt, rpt)])
        plsc.subcore_barrier()

        def step(j, carry):
            pltpu.async_copy(table.at[src_v.at[j]], rows_v, sem).wait()
            pltpu.sync_copy(rows_v, acc.at[dst_v.at[j]], add=True)
            return carry

        lax.fori_loop(0, n_chunks, step, 0)
        plsc.subcore_barrier()
        pltpu.sync_copy(acc.at[pl.ds(s * rpt, rpt)], out.at[c, pl.ds(s * rpt, rpt)])

    return pl.kernel(
        body,
        out_type=jax.ShapeDtypeStruct((N_CORES, n_pad, w), jnp.float32),
        mesh=mesh,
        scratch_types=[
            pltpu.VMEM((n_chunks, BLK), jnp.int32),
            pltpu.VMEM((n_chunks, BLK), jnp.int32),
            pltpu.VMEM((BLK, w), jnp.float32),
            pltpu.VMEM_SHARED((n_pad, w), jnp.float32),
            pltpu.SemaphoreType.DMA,
        ],
        compiler_params=pltpu.CompilerParams(use_tc_tiling_on_sc=False),
    )


def _tc_first(n, n_pad):
    def body(degp_ref, x_ref, w_ref, dinv_ref, h_ref):
        dp = degp_ref[...]
        deg = dp[0, :, 0] + dp[1, :, 0] + 1.0
        dinv = lax.rsqrt(deg)
        dinv_ref[...] = dinv[:, None]
        h = jnp.dot(x_ref[...], w_ref[...], preferred_element_type=jnp.float32)
        h_ref[...] = jnp.concatenate(
            [h * dinv[:n, None], jnp.zeros((n_pad - n, h.shape[1]), jnp.float32)], 0)
    return body


def _tc_mid(n):
    def body(p_ref, hp_ref, dinv_ref, b_ref, g_ref, be_ref, wn_ref, out_ref):
        p = p_ref[...]
        dinv = dinv_ref[...]
        u = (p[0] + p[1] + hp_ref[...]) * dinv + b_ref[...][None, :]
        n_pad = u.shape[0]
        mask = lax.broadcasted_iota(jnp.int32, (n_pad, 1), 0) < n
        u = jnp.where(mask, u, 0.0)
        mean = jnp.sum(u, axis=0, keepdims=True) / n
        var = jnp.sum(u * u, axis=0, keepdims=True) / n - mean * mean
        z = (u - mean) * lax.rsqrt(var + EPS) * g_ref[...][None, :] + be_ref[...][None, :]
        z = jnp.where(mask, jnp.maximum(z, 0.0), 0.0)
        out_ref[...] = jnp.dot(z, wn_ref[...], preferred_element_type=jnp.float32) * dinv
    return body


def _tc_last():
    def body(p_ref, hp_ref, dinv_ref, b_ref, out_ref):
        p = p_ref[...]
        out_ref[...] = (p[0] + p[1] + hp_ref[...]) * dinv_ref[...] + b_ref[...][None, :]
    return body


def _call_tc(body, out_shapes, *args):
    return pl.pallas_call(body, out_shape=out_shapes)(*args)


def kernel(x, edge_index, W1, b1, g1, beta1, W2, b2, g2, beta2,
           W3, b3, g3, beta3, W4, b4):
    n, d = x.shape
    h_dim = W1.shape[1]
    c_dim = W4.shape[1]
    e = edge_index.shape[1]

    n_chunks = -(-e // (N_TILES * BLK))
    e_pad = N_TILES * n_chunks * BLK
    n_pad = _ceil_to(n + 1, N_SUB * 8)
    wc = _ceil_to(c_dim, 64)

    src = edge_index[0].astype(jnp.int32)
    dst = edge_index[1].astype(jnp.int32)
    fill = jnp.full((e_pad - e,), n, jnp.int32)
    src3 = jnp.concatenate([src, fill]).reshape(N_TILES, n_chunks, BLK)
    dst3 = jnp.concatenate([dst, fill]).reshape(N_TILES, n_chunks, BLK)

    zeros_h = jnp.zeros((n_pad, h_dim), jnp.float32)
    zeros_c = jnp.zeros((n_pad, wc), jnp.float32)
    zeros_16 = jnp.zeros((n_pad, 16), jnp.float32)
    ones_16 = jnp.where(jnp.arange(n_pad)[:, None] < n,
                        jnp.float32(1), jnp.float32(0)) * jnp.ones((1, 16), jnp.float32)

    seg16 = _make_seg_sum(n_pad, n_chunks, 16)
    seg_h = _make_seg_sum(n_pad, n_chunks, h_dim)
    seg_c = _make_seg_sum(n_pad, n_chunks, wc)

    degp = seg16(ones_16, src3, dst3, zeros_16)

    f32 = jnp.float32
    dinv, h1 = _call_tc(
        _tc_first(n, n_pad),
        (jax.ShapeDtypeStruct((n_pad, 1), f32), jax.ShapeDtypeStruct((n_pad, h_dim), f32)),
        degp, x, W1)

    mid = _tc_mid(n)
    p1 = seg_h(h1, src3, dst3, zeros_h)
    h2 = _call_tc(mid, jax.ShapeDtypeStruct((n_pad, h_dim), f32),
                  p1, h1, dinv, b1, g1, beta1, W2)
    p2 = seg_h(h2, src3, dst3, zeros_h)
    h3 = _call_tc(mid, jax.ShapeDtypeStruct((n_pad, h_dim), f32),
                  p2, h2, dinv, b2, g2, beta2, W3)
    p3 = seg_h(h3, src3, dst3, zeros_h)
    W4p = jnp.pad(W4, ((0, 0), (0, wc - c_dim)))
    b4p = jnp.pad(b4, (0, wc - c_dim))
    h4 = _call_tc(mid, jax.ShapeDtypeStruct((n_pad, wc), f32),
                  p3, h3, dinv, b3, g3, beta3, W4p)
    p4 = seg_c(h4, src3, dst3, zeros_c)
    outp = _call_tc(_tc_last(), jax.ShapeDtypeStruct((n_pad, wc), f32),
                    p4, h4, dinv, b4p)
    return outp[:n, :c_dim]


# speedup vs baseline: 12.7192x; 1.2628x over previous
import functools

import jax
import jax.numpy as jnp
from jax import lax
from jax.experimental import pallas as pl
from jax.experimental.pallas import tpu as pltpu
from jax.experimental.pallas import tpu_sc as plsc

N_CORES = 2
N_SUB = 16
BLK = 128
EPS = 1e-5


def _ceil_to(v, m):
    return -(-v // m) * m


@functools.lru_cache(maxsize=None)
def _make_seg_sum(n_pad, n_chunks, w2):
    rpt = n_pad // N_SUB

    def body(table, src, dst, zeros, out, src_v, dst_v, rows_v, acc, sem):
        c = lax.axis_index("c")
        s = lax.axis_index("s")
        pltpu.sync_copy(src.at[s], src_v)
        pltpu.sync_copy(dst.at[s], dst_v)
        pltpu.sync_copy(zeros.at[pl.ds(s * rpt, rpt)], acc.at[pl.ds(s * rpt, rpt)])
        plsc.subcore_barrier()

        pltpu.async_copy(table.at[c].at[src_v.at[0]], rows_v.at[0], sem)

        def step(j, carry):
            slot = lax.rem(j, 2)
            pltpu.make_async_copy(table.at[c].at[src_v.at[j]], rows_v.at[slot], sem).wait()
            pltpu.async_copy(table.at[c].at[src_v.at[j + 1]], rows_v.at[1 - slot], sem)
            pltpu.sync_copy(rows_v.at[slot], acc.at[dst_v.at[j]], add=True)
            return carry

        lax.fori_loop(0, n_chunks - 1, step, 0)
        last = n_chunks - 1
        slot = lax.rem(last, 2)
        pltpu.make_async_copy(table.at[c].at[src_v.at[last]], rows_v.at[slot], sem).wait()
        pltpu.sync_copy(rows_v.at[slot], acc.at[dst_v.at[last]], add=True)
        plsc.subcore_barrier()
        pltpu.sync_copy(acc.at[pl.ds(s * rpt, rpt)], out.at[c, pl.ds(s * rpt, rpt)])

    return pl.kernel(
        body,
        out_type=jax.ShapeDtypeStruct((N_CORES, n_pad, w2), jnp.float32),
        mesh=plsc.VectorSubcoreMesh(core_axis_name="c", subcore_axis_name="s"),
        scratch_types=[
            pltpu.VMEM((n_chunks, BLK), jnp.int32),
            pltpu.VMEM((n_chunks, BLK), jnp.int32),
            pltpu.VMEM((2, BLK, w2), jnp.float32),
            pltpu.VMEM_SHARED((n_pad, w2), jnp.float32),
            pltpu.SemaphoreType.DMA,
        ],
        compiler_params=pltpu.CompilerParams(use_tc_tiling_on_sc=False),
    )


def _split(h):
    w2 = h.shape[1] // 2
    return jnp.stack([h[:, :w2], h[:, w2:]], 0)


def _tc_first(n, n_pad):
    def body(degp_ref, x_ref, w_ref, dinv_ref, h_ref):
        deg = degp_ref[0, :, 0] + 1.0
        dinv = lax.rsqrt(deg)
        dinv_ref[...] = dinv[:, None]
        h = jnp.dot(x_ref[...], w_ref[...], preferred_element_type=jnp.float32)
        hp = jnp.concatenate(
            [h * dinv[:n, None], jnp.zeros((n_pad - n, h.shape[1]), jnp.float32)], 0)
        h_ref[...] = _split(hp)
    return body


def _tc_mid(n):
    def body(p_ref, hp_ref, dinv_ref, b_ref, g_ref, be_ref, wn_ref, out_ref):
        p = p_ref[...]
        hp = hp_ref[...]
        dinv = dinv_ref[...]
        seg = jnp.concatenate([p[0] + hp[0], p[1] + hp[1]], 1)
        u = seg * dinv + b_ref[...][None, :]
        n_pad = u.shape[0]
        mask = lax.broadcasted_iota(jnp.int32, (n_pad, 1), 0) < n
        u = jnp.where(mask, u, 0.0)
        mean = jnp.sum(u, axis=0, keepdims=True) / n
        var = jnp.sum(u * u, axis=0, keepdims=True) / n - mean * mean
        z = (u - mean) * lax.rsqrt(var + EPS) * g_ref[...][None, :] + be_ref[...][None, :]
        z = jnp.where(mask, jnp.maximum(z, 0.0), 0.0)
        out_ref[...] = _split(
            jnp.dot(z, wn_ref[...], preferred_element_type=jnp.float32) * dinv)
    return body


def _tc_last():
    def body(p_ref, hp_ref, dinv_ref, b_ref, out_ref):
        p = p_ref[...]
        hp = hp_ref[...]
        seg = jnp.concatenate([p[0] + hp[0], p[1] + hp[1]], 1)
        out_ref[...] = seg * dinv_ref[...] + b_ref[...][None, :]
    return body


def _call_tc(body, out_shapes, *args):
    return pl.pallas_call(body, out_shape=out_shapes)(*args)


def kernel(x, edge_index, W1, b1, g1, beta1, W2, b2, g2, beta2,
           W3, b3, g3, beta3, W4, b4):
    n, d = x.shape
    h_dim = W1.shape[1]
    c_dim = W4.shape[1]
    e = edge_index.shape[1]

    n_chunks = -(-e // (N_SUB * BLK))
    e_pad = N_SUB * n_chunks * BLK
    n_pad = _ceil_to(n + 1, N_SUB * 8)
    wc = _ceil_to(c_dim, 64)

    src = edge_index[0].astype(jnp.int32)
    dst = edge_index[1].astype(jnp.int32)
    fill = jnp.full((e_pad - e,), n, jnp.int32)
    src3 = jnp.concatenate([src, fill]).reshape(N_SUB, n_chunks, BLK)
    dst3 = jnp.concatenate([dst, fill]).reshape(N_SUB, n_chunks, BLK)

    zeros_h2 = jnp.zeros((n_pad, h_dim // 2), jnp.float32)
    zeros_c2 = jnp.zeros((n_pad, wc // 2), jnp.float32)
    ones_c2 = jnp.where(jnp.arange(n_pad)[:, None] < n, jnp.float32(1),
                        jnp.float32(0)) * jnp.ones((1, wc // 2), jnp.float32)
    ones_tab = jnp.stack([ones_c2, ones_c2], 0)

    seg_h = _make_seg_sum(n_pad, n_chunks, h_dim // 2)
    seg_c = _make_seg_sum(n_pad, n_chunks, wc // 2)

    degp = seg_c(ones_tab, src3, dst3, zeros_c2)

    f32 = jnp.float32
    dinv, h1 = _call_tc(
        _tc_first(n, n_pad),
        (jax.ShapeDtypeStruct((n_pad, 1), f32),
         jax.ShapeDtypeStruct((2, n_pad, h_dim // 2), f32)),
        degp, x, W1)

    mid = _tc_mid(n)
    sh_h = jax.ShapeDtypeStruct((2, n_pad, h_dim // 2), f32)
    p1 = seg_h(h1, src3, dst3, zeros_h2)
    h2 = _call_tc(mid, sh_h, p1, h1, dinv, b1, g1, beta1, W2)
    p2 = seg_h(h2, src3, dst3, zeros_h2)
    h3 = _call_tc(mid, sh_h, p2, h2, dinv, b2, g2, beta2, W3)
    p3 = seg_h(h3, src3, dst3, zeros_h2)
    W4p = jnp.pad(W4, ((0, 0), (0, wc - c_dim)))
    b4p = jnp.pad(b4, (0, wc - c_dim))
    h4 = _call_tc(mid, jax.ShapeDtypeStruct((2, n_pad, wc // 2), f32),
                  p3, h3, dinv, b3, g3, beta3, W4p)
    p4 = seg_c(h4, src3, dst3, zeros_c2)
    outp = _call_tc(_tc_last(), jax.ShapeDtypeStruct((n_pad, wc), f32),
                    p4, h4, dinv, b4p)
    return outp[:n, :c_dim]


# speedup vs baseline: 12.9132x; 1.0153x over previous
import functools

import jax
import jax.numpy as jnp
from jax import lax
from jax.experimental import pallas as pl
from jax.experimental.pallas import tpu as pltpu
from jax.experimental.pallas import tpu_sc as plsc

N_CORES = 2
N_SUB = 16
BLK = 128
EPS = 1e-5


def _ceil_to(v, m):
    return -(-v // m) * m


@functools.lru_cache(maxsize=None)
def _make_seg_sum(n_pad, n_chunks, w2):
    rpt = n_pad // N_SUB

    def body(table, src, dst, zeros, out, src_v, dst_v, rows_v, acc, sem):
        c = lax.axis_index("c")
        s = lax.axis_index("s")
        pltpu.sync_copy(src.at[s], src_v)
        pltpu.sync_copy(dst.at[s], dst_v)
        pltpu.sync_copy(zeros.at[pl.ds(s * rpt, rpt)], acc.at[pl.ds(s * rpt, rpt)])
        plsc.subcore_barrier()

        pltpu.async_copy(table.at[c].at[src_v.at[0]], rows_v.at[0], sem)

        def step(j, carry):
            slot = lax.rem(j, 2)
            pltpu.make_async_copy(table.at[c].at[src_v.at[j]], rows_v.at[slot], sem).wait()
            pltpu.async_copy(table.at[c].at[src_v.at[j + 1]], rows_v.at[1 - slot], sem)
            pltpu.sync_copy(rows_v.at[slot], acc.at[dst_v.at[j]], add=True)
            return carry

        lax.fori_loop(0, n_chunks - 1, step, 0)
        last = n_chunks - 1
        slot = lax.rem(last, 2)
        pltpu.make_async_copy(table.at[c].at[src_v.at[last]], rows_v.at[slot], sem).wait()
        pltpu.sync_copy(rows_v.at[slot], acc.at[dst_v.at[last]], add=True)
        plsc.subcore_barrier()
        pltpu.sync_copy(acc.at[pl.ds(s * rpt, rpt)], out.at[c, pl.ds(s * rpt, rpt)])

    return pl.kernel(
        body,
        out_type=jax.ShapeDtypeStruct((N_CORES, n_pad, w2), jnp.float32),
        mesh=plsc.VectorSubcoreMesh(core_axis_name="c", subcore_axis_name="s"),
        scratch_types=[
            pltpu.VMEM((n_chunks, BLK), jnp.int32),
            pltpu.VMEM((n_chunks, BLK), jnp.int32),
            pltpu.VMEM((2, BLK, w2), jnp.float32),
            pltpu.VMEM_SHARED((n_pad, w2), jnp.float32),
            pltpu.SemaphoreType.DMA,
        ],
        compiler_params=pltpu.CompilerParams(use_tc_tiling_on_sc=False),
    )


def _split(h):
    w2 = h.shape[1] // 2
    return jnp.stack([h[:, :w2], h[:, w2:]], 0)


def _tc_first(n, n_pad):
    def body(degp_ref, x_ref, w_ref, dinv_ref, h_ref):
        deg = degp_ref[0, :, 0] + 1.0
        dinv = lax.rsqrt(deg)
        dinv_ref[...] = dinv[:, None]
        h = jnp.dot(x_ref[...], w_ref[...], preferred_element_type=jnp.float32)
        hp = jnp.concatenate(
            [h * dinv[:n, None], jnp.zeros((n_pad - n, h.shape[1]), jnp.float32)], 0)
        h_ref[...] = _split(hp)
    return body


def _tc_mid(n):
    def body(p_ref, hp_ref, dinv_ref, b_ref, g_ref, be_ref, wn_ref, out_ref):
        p = p_ref[...]
        hp = hp_ref[...]
        dinv = dinv_ref[...]
        seg = jnp.concatenate([p[0] + hp[0], p[1] + hp[1]], 1)
        u = seg * dinv + b_ref[...][None, :]
        n_pad = u.shape[0]
        mask = lax.broadcasted_iota(jnp.int32, (n_pad, 1), 0) < n
        u = jnp.where(mask, u, 0.0)
        mean = jnp.sum(u, axis=0, keepdims=True) / n
        var = jnp.sum(u * u, axis=0, keepdims=True) / n - mean * mean
        z = (u - mean) * lax.rsqrt(var + EPS) * g_ref[...][None, :] + be_ref[...][None, :]
        z = jnp.where(mask, jnp.maximum(z, 0.0), 0.0)
        out_ref[...] = _split(
            jnp.dot(z, wn_ref[...], preferred_element_type=jnp.float32) * dinv)
    return body


def _tc_last():
    def body(p_ref, hp_ref, dinv_ref, b_ref, out_ref):
        p = p_ref[...]
        hp = hp_ref[...]
        seg = jnp.concatenate([p[0] + hp[0], p[1] + hp[1]], 1)
        out_ref[...] = seg * dinv_ref[...] + b_ref[...][None, :]
    return body


def _call_tc(body, out_shapes, *args):
    return pl.pallas_call(body, out_shape=out_shapes)(*args)


def kernel(x, edge_index, W1, b1, g1, beta1, W2, b2, g2, beta2,
           W3, b3, g3, beta3, W4, b4):
    n, d = x.shape
    h_dim = W1.shape[1]
    c_dim = W4.shape[1]
    e = edge_index.shape[1]

    n_chunks = -(-e // (N_SUB * BLK))
    e_pad = N_SUB * n_chunks * BLK
    n_pad = _ceil_to(n + 1, N_SUB * 8)
    wc = _ceil_to(c_dim, 64)

    src = edge_index[0].astype(jnp.int32)
    dst = edge_index[1].astype(jnp.int32)
    fill = jnp.full((e_pad - e,), n, jnp.int32)
    src3 = jnp.concatenate([src, fill]).reshape(N_SUB, n_chunks, BLK)
    dst3 = jnp.concatenate([dst, fill]).reshape(N_SUB, n_chunks, BLK)

    zeros_h2 = jnp.zeros((n_pad, h_dim // 2), jnp.float32)
    zeros_c2 = jnp.zeros((n_pad, wc // 2), jnp.float32)
    zeros_16 = jnp.zeros((n_pad, 16), jnp.float32)
    ones_16 = jnp.where(jnp.arange(n_pad)[:, None] < n, jnp.float32(1),
                        jnp.float32(0)) * jnp.ones((1, 16), jnp.float32)
    ones_tab = jnp.stack([ones_16, ones_16], 0)

    seg_h = _make_seg_sum(n_pad, n_chunks, h_dim // 2)
    seg_c = _make_seg_sum(n_pad, n_chunks, wc // 2)
    seg_d = _make_seg_sum(n_pad, n_chunks, 16)

    degp = seg_d(ones_tab, src3, dst3, zeros_16)

    f32 = jnp.float32
    dinv, h1 = _call_tc(
        _tc_first(n, n_pad),
        (jax.ShapeDtypeStruct((n_pad, 1), f32),
         jax.ShapeDtypeStruct((2, n_pad, h_dim // 2), f32)),
        degp, x, W1)

    mid = _tc_mid(n)
    sh_h = jax.ShapeDtypeStruct((2, n_pad, h_dim // 2), f32)
    p1 = seg_h(h1, src3, dst3, zeros_h2)
    h2 = _call_tc(mid, sh_h, p1, h1, dinv, b1, g1, beta1, W2)
    p2 = seg_h(h2, src3, dst3, zeros_h2)
    h3 = _call_tc(mid, sh_h, p2, h2, dinv, b2, g2, beta2, W3)
    p3 = seg_h(h3, src3, dst3, zeros_h2)
    W4p = jnp.pad(W4, ((0, 0), (0, wc - c_dim)))
    b4p = jnp.pad(b4, (0, wc - c_dim))
    h4 = _call_tc(mid, jax.ShapeDtypeStruct((2, n_pad, wc // 2), f32),
                  p3, h3, dinv, b3, g3, beta3, W4p)
    p4 = seg_c(h4, src3, dst3, zeros_c2)
    outp = _call_tc(_tc_last(), jax.ShapeDtypeStruct((n_pad, wc), f32),
                    p4, h4, dinv, b4p)
    return outp[:n, :c_dim]


# speedup vs baseline: 17.9723x; 1.3918x over previous
import functools

import jax
import jax.numpy as jnp
from jax import lax
from jax.experimental import pallas as pl
from jax.experimental.pallas import tpu as pltpu
from jax.experimental.pallas import tpu_sc as plsc

N_CORES = 2
N_SUB = 16
BLK = 128
NBUF = 5
EPS = 1e-5


def _ceil_to(v, m):
    return -(-v // m) * m


@functools.lru_cache(maxsize=None)
def _make_seg_sum(n_pad, n_chunks, w2):
    rpt = n_pad // N_SUB

    def body(table, src, dst, zeros, out, src_v, dst_v, rows_v, acc, sem, sem2):
        c = lax.axis_index("c")
        s = lax.axis_index("s")
        pltpu.sync_copy(src.at[s], src_v)
        pltpu.sync_copy(dst.at[s], dst_v)
        pltpu.sync_copy(zeros.at[pl.ds(s * rpt, rpt)], acc.at[pl.ds(s * rpt, rpt)])
        plsc.subcore_barrier()

        def fire_gather(j):
            pltpu.async_copy(table.at[c].at[src_v.at[j]], rows_v.at[lax.rem(j, NBUF)], sem)

        def drain_one_scatter():
            pltpu.make_async_copy(rows_v.at[0], acc.at[dst_v.at[0]], sem2).wait()

        for j in range(3):
            fire_gather(j)

        def step(j, carry):
            pltpu.make_async_copy(
                table.at[c].at[src_v.at[j]], rows_v.at[lax.rem(j, NBUF)], sem).wait()
            pltpu.async_copy(rows_v.at[lax.rem(j, NBUF)], acc.at[dst_v.at[j]], sem2,
                             add=True)

            @pl.when(j >= 2)
            def _():
                drain_one_scatter()

            @pl.when(j + 3 < n_chunks)
            def _():
                fire_gather(j + 3)

            return carry

        lax.fori_loop(0, n_chunks, step, 0)
        drain_one_scatter()
        drain_one_scatter()
        plsc.subcore_barrier()
        pltpu.sync_copy(acc.at[pl.ds(s * rpt, rpt)], out.at[c, pl.ds(s * rpt, rpt)])

    return pl.kernel(
        body,
        out_type=jax.ShapeDtypeStruct((N_CORES, n_pad, w2), jnp.float32),
        mesh=plsc.VectorSubcoreMesh(core_axis_name="c", subcore_axis_name="s"),
        scratch_types=[
            pltpu.VMEM((n_chunks, BLK), jnp.int32),
            pltpu.VMEM((n_chunks, BLK), jnp.int32),
            pltpu.VMEM((NBUF, BLK, w2), jnp.float32),
            pltpu.VMEM_SHARED((n_pad, w2), jnp.float32),
            pltpu.SemaphoreType.DMA,
            pltpu.SemaphoreType.DMA,
        ],
        compiler_params=pltpu.CompilerParams(use_tc_tiling_on_sc=False),
    )


def _split(h):
    w2 = h.shape[1] // 2
    return jnp.stack([h[:, :w2], h[:, w2:]], 0)


def _tc_first(n, n_pad):
    def body(degp_ref, x_ref, w_ref, dinv_ref, h_ref):
        deg = degp_ref[0, :, 0] + 1.0
        dinv = lax.rsqrt(deg)
        dinv_ref[...] = dinv[:, None]
        h = jnp.dot(x_ref[...], w_ref[...], preferred_element_type=jnp.float32)
        hp = jnp.concatenate(
            [h * dinv[:n, None], jnp.zeros((n_pad - n, h.shape[1]), jnp.float32)], 0)
        h_ref[...] = _split(hp)
    return body


def _tc_mid(n):
    def body(p_ref, hp_ref, dinv_ref, b_ref, g_ref, be_ref, wn_ref, out_ref):
        p = p_ref[...]
        hp = hp_ref[...]
        dinv = dinv_ref[...]
        seg = jnp.concatenate([p[0] + hp[0], p[1] + hp[1]], 1)
        u = seg * dinv + b_ref[...][None, :]
        n_pad = u.shape[0]
        mask = lax.broadcasted_iota(jnp.int32, (n_pad, 1), 0) < n
        u = jnp.where(mask, u, 0.0)
        mean = jnp.sum(u, axis=0, keepdims=True) / n
        var = jnp.sum(u * u, axis=0, keepdims=True) / n - mean * mean
        z = (u - mean) * lax.rsqrt(var + EPS) * g_ref[...][None, :] + be_ref[...][None, :]
        z = jnp.where(mask, jnp.maximum(z, 0.0), 0.0)
        out_ref[...] = _split(
            jnp.dot(z, wn_ref[...], preferred_element_type=jnp.float32) * dinv)
    return body


def _tc_last():
    def body(p_ref, hp_ref, dinv_ref, b_ref, out_ref):
        p = p_ref[...]
        hp = hp_ref[...]
        seg = jnp.concatenate([p[0] + hp[0], p[1] + hp[1]], 1)
        out_ref[...] = seg * dinv_ref[...] + b_ref[...][None, :]
    return body


def _call_tc(body, out_shapes, *args):
    return pl.pallas_call(body, out_shape=out_shapes)(*args)


def kernel(x, edge_index, W1, b1, g1, beta1, W2, b2, g2, beta2,
           W3, b3, g3, beta3, W4, b4):
    n, d = x.shape
    h_dim = W1.shape[1]
    c_dim = W4.shape[1]
    e = edge_index.shape[1]

    n_chunks = -(-e // (N_SUB * BLK))
    e_pad = N_SUB * n_chunks * BLK
    n_pad = _ceil_to(n + 1, N_SUB * 8)
    wc = _ceil_to(c_dim, 64)

    src = edge_index[0].astype(jnp.int32)
    dst = edge_index[1].astype(jnp.int32)
    fill = jnp.full((e_pad - e,), n, jnp.int32)
    src3 = jnp.concatenate([src, fill]).reshape(N_SUB, n_chunks, BLK)
    dst3 = jnp.concatenate([dst, fill]).reshape(N_SUB, n_chunks, BLK)

    zeros_h2 = jnp.zeros((n_pad, h_dim // 2), jnp.float32)
    zeros_c2 = jnp.zeros((n_pad, wc // 2), jnp.float32)
    zeros_16 = jnp.zeros((n_pad, 16), jnp.float32)
    ones_16 = jnp.where(jnp.arange(n_pad)[:, None] < n, jnp.float32(1),
                        jnp.float32(0)) * jnp.ones((1, 16), jnp.float32)
    ones_tab = jnp.stack([ones_16, ones_16], 0)

    seg_h = _make_seg_sum(n_pad, n_chunks, h_dim // 2)
    seg_c = _make_seg_sum(n_pad, n_chunks, wc // 2)
    seg_d = _make_seg_sum(n_pad, n_chunks, 16)

    degp = seg_d(ones_tab, src3, dst3, zeros_16)

    f32 = jnp.float32
    dinv, h1 = _call_tc(
        _tc_first(n, n_pad),
        (jax.ShapeDtypeStruct((n_pad, 1), f32),
         jax.ShapeDtypeStruct((2, n_pad, h_dim // 2), f32)),
        degp, x, W1)

    mid = _tc_mid(n)
    sh_h = jax.ShapeDtypeStruct((2, n_pad, h_dim // 2), f32)
    p1 = seg_h(h1, src3, dst3, zeros_h2)
    h2 = _call_tc(mid, sh_h, p1, h1, dinv, b1, g1, beta1, W2)
    p2 = seg_h(h2, src3, dst3, zeros_h2)
    h3 = _call_tc(mid, sh_h, p2, h2, dinv, b2, g2, beta2, W3)
    p3 = seg_h(h3, src3, dst3, zeros_h2)
    W4p = jnp.pad(W4, ((0, 0), (0, wc - c_dim)))
    b4p = jnp.pad(b4, (0, wc - c_dim))
    h4 = _call_tc(mid, jax.ShapeDtypeStruct((2, n_pad, wc // 2), f32),
                  p3, h3, dinv, b3, g3, beta3, W4p)
    p4 = seg_c(h4, src3, dst3, zeros_c2)
    outp = _call_tc(_tc_last(), jax.ShapeDtypeStruct((n_pad, wc), f32),
                    p4, h4, dinv, b4p)
    return outp[:n, :c_dim]
